# Initial kernel scaffold; baseline (speedup 1.0000x reference)
import jax
import jax.numpy as jnp
from jax.experimental import pallas as pl


def kernel(features, edge_index, W0, b0, W1, b1, W2, b2):
    raise NotImplementedError("write your pallas kernel here")


# speedup vs baseline: 6.1294x; 6.1294x over previous
import jax
import jax.numpy as jnp
from jax import lax
from jax.experimental import pallas as pl
from jax.experimental.pallas import tpu as pltpu
from jax.experimental.pallas import tpu_sc as plsc

N = 100000
E = 1600000
NP = 100096
DQ = 10
NQ = 4
ALPHA = 0.1
K_PROP = 10

N_SUBCORES = 16
ROWS_PER_SUB = NP // N_SUBCORES
ROW_CHUNK = 391
N_ROW_CHUNKS = ROWS_PER_SUB // ROW_CHUNK
EDGES_PER_SUB = E // N_SUBCORES
EDGE_CHUNK = 2000
N_EDGE_CHUNKS = EDGES_PER_SUB // EDGE_CHUNK
DEG_PER_SUB = NP // N_SUBCORES

_SC_MESH = dict(core_axis_name="c", subcore_axis_name="s",
                num_cores=2, num_subcores=16)
_SC_PARAMS = pltpu.CompilerParams(use_tc_tiling_on_sc=False)


def _mlp_body(x, w0, b0, w1, b1, w2, b2, no, ci, *outs):
    h = jax.nn.relu(
        jnp.dot(x[...], w0[...], preferred_element_type=jnp.float32) + b0[...]
    )
    h = jax.nn.relu(
        jnp.dot(h, w1[...], preferred_element_type=jnp.float32) + b1[...]
    )
    h = jnp.dot(h, w2[...], preferred_element_type=jnp.float32) + b2[...]
    g = h * no[...]
    c = h * ci[...]
    for q in range(NQ):
        outs[q][...] = g[:, q * DQ:(q + 1) * DQ]
        outs[NQ + q][...] = c[:, q * DQ:(q + 1) * DQ]


def _mlp(x, w0, b0, w1, b1, w2, b2, no, ci):
    r = 1000
    quarter = jax.ShapeDtypeStruct((NP, DQ), jnp.float32)
    qspec = pl.BlockSpec((r, DQ), lambda i: (i, 0))
    return pl.pallas_call(
        _mlp_body,
        grid=(N // r,),
        in_specs=[
            pl.BlockSpec((r, 128), lambda i: (i, 0)),
            pl.BlockSpec((128, 128), lambda i: (0, 0)),
            pl.BlockSpec((1, 128), lambda i: (0, 0)),
            pl.BlockSpec((128, 128), lambda i: (0, 0)),
            pl.BlockSpec((1, 128), lambda i: (0, 0)),
            pl.BlockSpec((128, NQ * DQ), lambda i: (0, 0)),
            pl.BlockSpec((1, NQ * DQ), lambda i: (0, 0)),
            pl.BlockSpec((r, 1), lambda i: (i, 0)),
            pl.BlockSpec((r, 1), lambda i: (i, 0)),
        ],
        out_specs=[qspec] * (2 * NQ),
        out_shape=[quarter] * (2 * NQ),
    )(x, w0, b0, w1, b1, w2, b2, no, ci)


def _scale_body(a0, a1, a2, a3, p, o0, o1, o2, o3):
    o0[...] = a0[...] * p[...]
    o1[...] = a1[...] * p[...]
    o2[...] = a2[...] * p[...]
    o3[...] = a3[...] * p[...]


def _scale(accs, p):
    r = 3128
    quarter = jax.ShapeDtypeStruct((NP, DQ), jnp.float32)
    qspec = pl.BlockSpec((r, DQ), lambda i: (i, 0))
    return pl.pallas_call(
        _scale_body,
        grid=(NP // r,),
        in_specs=[qspec] * NQ + [pl.BlockSpec((r, 1), lambda i: (i, 0))],
        out_specs=[qspec] * NQ,
        out_shape=[quarter] * NQ,
    )(*accs, p)


def _degrees_body(src, dst, zeros, ones, dsrc, ddst, dacc, idx_v, ones_v,
                  dbuf_v, sem):
    c = lax.axis_index("c")
    s = lax.axis_index("s")

    def run(idx_hbm, out_hbm):
        d0 = pl.multiple_of(s * DEG_PER_SUB, 8)
        pltpu.sync_copy(zeros, dbuf_v)
        pltpu.sync_copy(dbuf_v, dacc.at[pl.ds(d0, DEG_PER_SUB)])
        plsc.subcore_barrier()
        pltpu.sync_copy(ones, ones_v)

        def step(j, carry):
            eb = pl.multiple_of(s * EDGES_PER_SUB + j * EDGE_CHUNK, 8)
            pltpu.sync_copy(idx_hbm.at[pl.ds(eb, EDGE_CHUNK)], idx_v)
            pltpu.sync_copy(ones_v, dacc.at[idx_v], add=True)
            return carry

        lax.fori_loop(0, N_EDGE_CHUNKS, step, 0)
        plsc.subcore_barrier()
        pltpu.sync_copy(dacc.at[pl.ds(d0, DEG_PER_SUB)], dbuf_v)
        pltpu.sync_copy(dbuf_v, out_hbm.at[pl.ds(d0, DEG_PER_SUB)])

    @pl.when(c == 0)
    def _():
        run(src, dsrc)

    @pl.when(c == 1)
    def _():
        run(dst, ddst)


def _degrees(src, dst, zeros, ones):
    deg = jax.ShapeDtypeStruct((NP,), jnp.float32)
    f = pl.kernel(
        _degrees_body,
        out_type=[deg, deg],
        mesh=plsc.VectorSubcoreMesh(**_SC_MESH),
        scratch_types=[
            pltpu.VMEM_SHARED((NP,), jnp.float32),
            pltpu.VMEM((EDGE_CHUNK,), jnp.int32),
            pltpu.VMEM((EDGE_CHUNK,), jnp.float32),
            pltpu.VMEM((DEG_PER_SUB,), jnp.float32),
            pltpu.SemaphoreType.DMA,
        ],
        compiler_params=_SC_PARAMS,
    )
    return f(src, dst, zeros, ones)


def _prop_body(ca, cb, ga, gb, src, dst, oa, ob,
               acc, src_v, dst_v, rows_v, bounce_v, sem):
    c = lax.axis_index("c")
    s = lax.axis_index("s")

    def run(ctbl, gtbl, out_hbm):
        def pre(k, carry):
            r0 = s * ROWS_PER_SUB + k * ROW_CHUNK
            pltpu.sync_copy(ctbl.at[pl.ds(r0, ROW_CHUNK)], bounce_v)
            pltpu.sync_copy(bounce_v, acc.at[pl.ds(r0, ROW_CHUNK)])
            return carry

        lax.fori_loop(0, N_ROW_CHUNKS, pre, 0)
        plsc.subcore_barrier()

        def step(j, carry):
            eb = pl.multiple_of(s * EDGES_PER_SUB + j * EDGE_CHUNK, 8)
            pltpu.sync_copy(src.at[pl.ds(eb, EDGE_CHUNK)], src_v)
            pltpu.sync_copy(dst.at[pl.ds(eb, EDGE_CHUNK)], dst_v)
            pltpu.async_copy(gtbl.at[src_v], rows_v, sem).wait()
            pltpu.sync_copy(rows_v, acc.at[dst_v], add=True)
            return carry

        lax.fori_loop(0, N_EDGE_CHUNKS, step, 0)
        plsc.subcore_barrier()

        def out(k, carry):
            r0 = s * ROWS_PER_SUB + k * ROW_CHUNK
            pltpu.sync_copy(acc.at[pl.ds(r0, ROW_CHUNK)], bounce_v)
            pltpu.sync_copy(bounce_v, out_hbm.at[pl.ds(r0, ROW_CHUNK)])
            return carry

        lax.fori_loop(0, N_ROW_CHUNKS, out, 0)

    @pl.when(c == 0)
    def _():
        run(ca, ga, oa)

    @pl.when(c == 1)
    def _():
        run(cb, gb, ob)


def _prop(c_a, c_b, g_a, g_b, src, dst):
    quarter = jax.ShapeDtypeStruct((NP, DQ), jnp.float32)
    f = pl.kernel(
        _prop_body,
        out_type=[quarter, quarter],
        mesh=plsc.VectorSubcoreMesh(**_SC_MESH),
        scratch_types=[
            pltpu.VMEM_SHARED((NP, DQ), jnp.float32),
            pltpu.VMEM((EDGE_CHUNK,), jnp.int32),
            pltpu.VMEM((EDGE_CHUNK,), jnp.int32),
            pltpu.VMEM((EDGE_CHUNK, DQ), jnp.float32),
            pltpu.VMEM((ROW_CHUNK, DQ), jnp.float32),
            pltpu.SemaphoreType.DMA,
        ],
        compiler_params=_SC_PARAMS,
    )
    return f(c_a, c_b, g_a, g_b, src, dst)


def kernel(features, edge_index, W0, b0, W1, b1, W2, b2):
    src = edge_index[0].astype(jnp.int32)
    dst = edge_index[1].astype(jnp.int32)

    zeros = jnp.zeros((DEG_PER_SUB,), jnp.float32)
    ones = jnp.ones((EDGE_CHUNK,), jnp.float32)
    deg_out_p, deg_in_p = _degrees(src, dst, zeros, ones)

    deg_out = jnp.maximum(deg_out_p[:N], 1.0)
    deg_in = jnp.maximum(deg_in_p[:N], 1.0)
    norm_out = lax.rsqrt(deg_out)
    norm_in = lax.rsqrt(deg_in)

    pad = ((0, NP - N),)
    p_g = jnp.pad((1.0 - ALPHA) * norm_in * norm_out, pad)[:, None]
    p_h = jnp.pad((1.0 - ALPHA) * norm_in, pad)[:, None]
    cinv = (ALPHA / (1.0 - ALPHA) / norm_in)[:, None]

    outs = _mlp(
        features, W0, b0.reshape(1, -1), W1, b1.reshape(1, -1),
        W2, b2.reshape(1, -1), norm_out[:, None], cinv,
    )
    g = list(outs[:NQ])
    c = list(outs[NQ:])

    for step in range(K_PROP):
        a0, a1 = _prop(c[0], c[1], g[0], g[1], src, dst)
        a2, a3 = _prop(c[2], c[3], g[2], g[3], src, dst)
        p = p_g if step < K_PROP - 1 else p_h
        g = _scale((a0, a1, a2, a3), p)

    return jnp.concatenate([q[:N] for q in g], axis=1)
